# baseline (speedup 1.0000x reference)
import functools

import jax
import jax.numpy as jnp
from jax import lax
from jax.experimental import pallas as pl
from jax.experimental.pallas import tpu as pltpu
from jax.experimental.pallas import tpu_sc as plsc

N = 10000
E = 320000
D = 128
G = 64
NC = 2
NS = 16
NW = NC * NS
CHUNK = 64
EPT = E // NW
NCHUNK = 160
EPT_PAD = NCHUNK * CHUNK
N_ACC = 10112
RPT = N_ACC // NS
RB = 2000

_mesh = plsc.VectorSubcoreMesh(core_axis_name="c", subcore_axis_name="s")


HALF = NCHUNK // 2
TOTCH = NW * NCHUNK
FAST_CID = 0
FCH = 320
SCH = (TOTCH - NS * FCH) // NS


def _phases(total):
    out = [HALF] * (total // HALF)
    if total % HALF:
        out.append(total % HALF)
    return out


def _pipelined_pass(hp_hbm, src_flat, dst_flat, acc_sh,
                    src_v, dst_v, rows_v, sems, tile_base, counts):
    off = 0
    for cnt in counts:
        pltpu.sync_copy(src_flat.at[pl.ds(tile_base + off, cnt)],
                        src_v.at[pl.ds(0, cnt)])
        pltpu.sync_copy(dst_flat.at[pl.ds(tile_base + off, cnt)],
                        dst_v.at[pl.ds(0, cnt)])

        def gstart(c, b):
            pltpu.async_copy(hp_hbm.at[src_v.at[c]], rows_v.at[b], sems[b])

        def gwait(c, b):
            pltpu.make_async_copy(
                hp_hbm.at[src_v.at[c]], rows_v.at[b], sems[b]).wait()

        for j in range(min(2, cnt)):
            gstart(j, j)

        def body(k, carry, cnt=cnt):
            for j in range(3):
                c = 3 * k + j
                gwait(c, j % 3)

                @pl.when(c + 2 < cnt)
                def _(c=c, j=j):
                    gstart(c + 2, (j + 2) % 3)

                pltpu.sync_copy(rows_v.at[j % 3], acc_sh.at[dst_v.at[c]],
                                add=True)
            return carry

        lax.fori_loop(0, cnt // 3, body, 0)
        for c in range(3 * (cnt // 3), cnt):
            gwait(c, c % 3)
            pltpu.sync_copy(rows_v.at[c % 3], acc_sh.at[dst_v.at[c]],
                            add=True)
        off += cnt


@functools.partial(
    pl.kernel,
    out_type=jax.ShapeDtypeStruct((NC, N_ACC, D), jnp.float32),
    mesh=_mesh,
    scratch_types=[
        pltpu.VMEM((HALF, CHUNK), jnp.int32),
        pltpu.VMEM((HALF, CHUNK), jnp.int32),
        pltpu.VMEM((3, CHUNK, D), jnp.float32),
        pltpu.VMEM_SHARED((N_ACC, D), jnp.float32),
        pltpu.SemaphoreType.DMA,
        pltpu.SemaphoreType.DMA,
        pltpu.SemaphoreType.DMA,
    ],
)
def _sc_aggregate(hp_hbm, hp2_hbm, src_hbm, dst_hbm, zero_hbm, out_hbm,
                  src_v, dst_v, rows_v, acc_sh, sem0, sem1, sem2):
    cid = lax.axis_index("c")
    sid = lax.axis_index("s")
    sems = (sem0, sem1, sem2)
    pltpu.sync_copy(zero_hbm.at[pl.ds(sid * RPT, RPT)],
                    acc_sh.at[pl.ds(sid * RPT, RPT)])
    plsc.subcore_barrier()

    @pl.when(cid == FAST_CID)
    def _():
        _pipelined_pass(hp_hbm, src_hbm, dst_hbm, acc_sh, src_v, dst_v,
                        rows_v, sems, sid * FCH, _phases(FCH))

    @pl.when(cid != FAST_CID)
    def _():
        _pipelined_pass(hp2_hbm, src_hbm, dst_hbm, acc_sh, src_v, dst_v,
                        rows_v, sems, NS * FCH + sid * SCH, _phases(SCH))

    plsc.subcore_barrier()
    pltpu.sync_copy(acc_sh.at[pl.ds(sid * RPT, RPT)],
                    out_hbm.at[cid].at[pl.ds(sid * RPT, RPT)])


@functools.partial(
    pl.kernel,
    out_type=jax.ShapeDtypeStruct((NC, N_ACC, D), jnp.float32),
    mesh=_mesh,
    scratch_types=[
        pltpu.VMEM((NCHUNK, CHUNK), jnp.int32),
        pltpu.VMEM((CHUNK, D), jnp.float32),
        pltpu.VMEM_SHARED((N_ACC, D), jnp.float32),
    ],
)
def _sc_degree(dst_hbm, zero_hbm, ones_hbm, out_hbm, dst_v, ones_v, acc_sh):
    cid = lax.axis_index("c")
    sid = lax.axis_index("s")
    wid = cid * NS + sid
    pltpu.sync_copy(dst_hbm.at[pl.ds(wid * NCHUNK, NCHUNK)], dst_v)
    pltpu.sync_copy(ones_hbm, ones_v)
    pltpu.sync_copy(zero_hbm.at[pl.ds(sid * RPT, RPT)],
                    acc_sh.at[pl.ds(sid * RPT, RPT)])
    plsc.subcore_barrier()

    def body(c, carry):
        pltpu.sync_copy(ones_v, acc_sh.at[dst_v.at[c]], add=True)
        return carry

    lax.fori_loop(0, NCHUNK, body, 0)

    plsc.subcore_barrier()
    pltpu.sync_copy(acc_sh.at[pl.ds(sid * RPT, RPT)],
                    out_hbm.at[cid].at[pl.ds(sid * RPT, RPT)])


def _elu(t):
    return jnp.where(t > 0, t, jnp.exp(t) - 1.0)


def _tc_pre(x, W1, dega, degb):
    def body(x_ref, w_ref, da_ref, db_ref, hp_ref, hp2_ref, dinv_ref):
        deg = da_ref[...] + db_ref[...] + 1.0
        dinv = lax.rsqrt(deg)
        z = jnp.dot(x_ref[...], w_ref[...], preferred_element_type=jnp.float32)
        hp = z * dinv
        hp_ref[...] = hp
        hp2_ref[...] = hp
        dinv_ref[...] = dinv

    return pl.pallas_call(
        body,
        grid=(N // RB,),
        in_specs=[
            pl.BlockSpec((RB, D), lambda i: (i, 0)),
            pl.BlockSpec((D, D), lambda i: (0, 0)),
            pl.BlockSpec((RB, 1), lambda i: (i, 0)),
            pl.BlockSpec((RB, 1), lambda i: (i, 0)),
        ],
        out_specs=[
            pl.BlockSpec((RB, D), lambda i: (i, 0)),
            pl.BlockSpec((RB, D), lambda i: (i, 0)),
            pl.BlockSpec((RB, 1), lambda i: (i, 0)),
        ],
        out_shape=[
            jax.ShapeDtypeStruct((N, D), jnp.float32),
            jax.ShapeDtypeStruct((N, D), jnp.float32),
            jax.ShapeDtypeStruct((N, 1), jnp.float32),
        ],
    )(x, W1, dega, degb)


def _tc_mid(agg0, agg1, hp, dinv, b, W):
    def body(a0_ref, a1_ref, hp_ref, dinv_ref, b_ref, w_ref,
             out_ref, out2_ref):
        dinv = dinv_ref[...]
        t = dinv * (a0_ref[...] + a1_ref[...] + hp_ref[...]) + b_ref[...]
        t = _elu(t)
        o = jnp.dot(t, w_ref[...], preferred_element_type=jnp.float32) * dinv
        out_ref[...] = o
        out2_ref[...] = o

    return pl.pallas_call(
        body,
        grid=(N // RB,),
        in_specs=[
            pl.BlockSpec((RB, D), lambda i: (i, 0)),
            pl.BlockSpec((RB, D), lambda i: (i, 0)),
            pl.BlockSpec((RB, D), lambda i: (i, 0)),
            pl.BlockSpec((RB, 1), lambda i: (i, 0)),
            pl.BlockSpec((1, D), lambda i: (0, 0)),
            pl.BlockSpec((D, D), lambda i: (0, 0)),
        ],
        out_specs=[
            pl.BlockSpec((RB, D), lambda i: (i, 0)),
            pl.BlockSpec((RB, D), lambda i: (i, 0)),
        ],
        out_shape=[
            jax.ShapeDtypeStruct((N, D), jnp.float32),
            jax.ShapeDtypeStruct((N, D), jnp.float32),
        ],
    )(agg0, agg1, hp, dinv, b, W)


def _tc_post(agg0, agg1, hp, dinv, b3, batch_col, Wl1, bl1, Wl2p, bl2p):
    nblk = N // RB

    def body(a0_ref, a1_ref, hp_ref, dinv_ref, b_ref, batch_ref,
             wl1_ref, bl1_ref, wl2_ref, bl2_ref, out_ref, pooled):
        i = pl.program_id(0)
        dinv = dinv_ref[...]
        t = dinv * (a0_ref[...] + a1_ref[...] + hp_ref[...]) + b_ref[...]
        t = _elu(t)
        seg = lax.broadcasted_iota(jnp.int32, (RB, G), 1)
        oh = (batch_ref[...] == seg).astype(jnp.float32)
        contrib = lax.dot_general(
            oh, t, (((0,), (0,)), ((), ())),
            preferred_element_type=jnp.float32)

        @pl.when(i == 0)
        def _():
            pooled[...] = contrib

        @pl.when(i > 0)
        def _():
            pooled[...] += contrib

        @pl.when(i == nblk - 1)
        def _():
            p = pooled[...]
            o1 = jnp.dot(p, wl1_ref[...],
                         preferred_element_type=jnp.float32) + bl1_ref[...]
            out_ref[...] = jnp.dot(
                o1, wl2_ref[...], preferred_element_type=jnp.float32) + bl2_ref[...]

    return pl.pallas_call(
        body,
        grid=(nblk,),
        in_specs=[
            pl.BlockSpec((RB, D), lambda i: (i, 0)),
            pl.BlockSpec((RB, D), lambda i: (i, 0)),
            pl.BlockSpec((RB, D), lambda i: (i, 0)),
            pl.BlockSpec((RB, 1), lambda i: (i, 0)),
            pl.BlockSpec((1, D), lambda i: (0, 0)),
            pl.BlockSpec((RB, 1), lambda i: (i, 0)),
            pl.BlockSpec((D, D), lambda i: (0, 0)),
            pl.BlockSpec((1, D), lambda i: (0, 0)),
            pl.BlockSpec((D, D), lambda i: (0, 0)),
            pl.BlockSpec((1, D), lambda i: (0, 0)),
        ],
        out_specs=pl.BlockSpec((G, D), lambda i: (0, 0)),
        out_shape=jax.ShapeDtypeStruct((G, D), jnp.float32),
        scratch_shapes=[pltpu.VMEM((G, D), jnp.float32)],
    )(agg0, agg1, hp, dinv, b3, batch_col, Wl1, bl1, Wl2p, bl2p)


def kernel(x, edge_index, batch, W1, b1, W2, b2, W3, b3, Wl1, bl1, Wl2, bl2):
    src = edge_index[0].astype(jnp.int32)
    dst = edge_index[1].astype(jnp.int32)
    pad = EPT_PAD * NW - E
    src_t = jnp.concatenate([src, jnp.zeros((pad,), jnp.int32)]).reshape(
        TOTCH, CHUNK)
    dst_t = jnp.concatenate([dst, jnp.full((pad,), N, jnp.int32)]).reshape(
        TOTCH, CHUNK)

    zeros_feat = jnp.zeros((N_ACC, D), jnp.float32)
    ones_rows = jnp.ones((CHUNK, D), jnp.float32)

    degp = _sc_degree(dst_t, zeros_feat, ones_rows)
    dega = degp[0, :N, 0:1]
    degb = degp[1, :N, 0:1]

    h1p, h1p2, dinv = _tc_pre(x, W1, dega, degb)

    agg = _sc_aggregate(h1p, h1p2, src_t, dst_t, zeros_feat)
    h2p, h2p2 = _tc_mid(agg[0, :N], agg[1, :N], h1p, dinv,
                        b1.reshape(1, D), W2)

    agg = _sc_aggregate(h2p, h2p2, src_t, dst_t, zeros_feat)
    h3p, h3p2 = _tc_mid(agg[0, :N], agg[1, :N], h2p, dinv,
                        b2.reshape(1, D), W3)

    agg = _sc_aggregate(h3p, h3p2, src_t, dst_t, zeros_feat)

    batch_col = batch.astype(jnp.int32).reshape(N, 1)
    Wl2p = jnp.zeros((D, D), jnp.float32).at[:, :2].set(Wl2)
    bl2p = jnp.zeros((1, D), jnp.float32).at[0, :2].set(bl2)
    out = _tc_post(agg[0, :N], agg[1, :N], h3p, dinv, b3.reshape(1, D),
                   batch_col, Wl1, bl1.reshape(1, D), Wl2p, bl2p)
    return out[:, :2]


# speedup vs baseline: 1.5653x; 1.5653x over previous
import functools

import jax
import jax.numpy as jnp
from jax import lax
from jax.experimental import pallas as pl
from jax.experimental.pallas import tpu as pltpu
from jax.experimental.pallas import tpu_sc as plsc

N = 10000
E = 320000
D = 128
G = 64
NC = 2
NS = 16
NW = NC * NS
CHUNK = 64
EPT = E // NW
NCHUNK = 160
EPT_PAD = NCHUNK * CHUNK
N_ACC = 10112
RPT = N_ACC // NS
RB = 2000

_mesh = plsc.VectorSubcoreMesh(core_axis_name="c", subcore_axis_name="s")


HALF = NCHUNK // 2
TOTCH = NW * NCHUNK
FAST_CID = 0
FCH = 304
SCH = (TOTCH - NS * FCH) // NS


def _phases(total):
    out = [HALF] * (total // HALF)
    if total % HALF:
        out.append(total % HALF)
    return out


def _pipelined_pass(hp_hbm, src_flat, dst_flat, acc_sh,
                    src_v, dst_v, rows_v, sems, tile_base, counts):
    off = 0
    for cnt in counts:
        pltpu.sync_copy(src_flat.at[pl.ds(tile_base + off, cnt)],
                        src_v.at[pl.ds(0, cnt)])
        pltpu.sync_copy(dst_flat.at[pl.ds(tile_base + off, cnt)],
                        dst_v.at[pl.ds(0, cnt)])

        def gstart(c, b):
            pltpu.async_copy(hp_hbm.at[src_v.at[c]], rows_v.at[b], sems[b])

        def gwait(c, b):
            pltpu.make_async_copy(
                hp_hbm.at[src_v.at[c]], rows_v.at[b], sems[b]).wait()

        for j in range(min(2, cnt)):
            gstart(j, j)

        def body(k, carry, cnt=cnt):
            for j in range(3):
                c = 3 * k + j
                gwait(c, j % 3)

                @pl.when(c + 2 < cnt)
                def _(c=c, j=j):
                    gstart(c + 2, (j + 2) % 3)

                pltpu.sync_copy(rows_v.at[j % 3], acc_sh.at[dst_v.at[c]],
                                add=True)
            return carry

        lax.fori_loop(0, cnt // 3, body, 0)
        for c in range(3 * (cnt // 3), cnt):
            gwait(c, c % 3)
            pltpu.sync_copy(rows_v.at[c % 3], acc_sh.at[dst_v.at[c]],
                            add=True)
        off += cnt


@functools.partial(
    pl.kernel,
    out_type=jax.ShapeDtypeStruct((NC, N_ACC, D), jnp.float32),
    mesh=_mesh,
    scratch_types=[
        pltpu.VMEM((HALF, CHUNK), jnp.int32),
        pltpu.VMEM((HALF, CHUNK), jnp.int32),
        pltpu.VMEM((3, CHUNK, D), jnp.float32),
        pltpu.VMEM_SHARED((N_ACC, D), jnp.float32),
        pltpu.SemaphoreType.DMA,
        pltpu.SemaphoreType.DMA,
        pltpu.SemaphoreType.DMA,
    ],
)
def _sc_aggregate(hp_hbm, hp2_hbm, src_hbm, dst_hbm, zero_hbm, out_hbm,
                  src_v, dst_v, rows_v, acc_sh, sem0, sem1, sem2):
    cid = lax.axis_index("c")
    sid = lax.axis_index("s")
    sems = (sem0, sem1, sem2)
    pltpu.sync_copy(zero_hbm.at[pl.ds(sid * RPT, RPT)],
                    acc_sh.at[pl.ds(sid * RPT, RPT)])
    plsc.subcore_barrier()

    @pl.when(cid == FAST_CID)
    def _():
        _pipelined_pass(hp_hbm, src_hbm, dst_hbm, acc_sh, src_v, dst_v,
                        rows_v, sems, sid * FCH, _phases(FCH))

    @pl.when(cid != FAST_CID)
    def _():
        _pipelined_pass(hp2_hbm, src_hbm, dst_hbm, acc_sh, src_v, dst_v,
                        rows_v, sems, NS * FCH + sid * SCH, _phases(SCH))

    plsc.subcore_barrier()
    pltpu.sync_copy(acc_sh.at[pl.ds(sid * RPT, RPT)],
                    out_hbm.at[cid].at[pl.ds(sid * RPT, RPT)])


@functools.partial(
    pl.kernel,
    out_type=jax.ShapeDtypeStruct((NC, N_ACC, D), jnp.float32),
    mesh=_mesh,
    scratch_types=[
        pltpu.VMEM((NCHUNK, CHUNK), jnp.int32),
        pltpu.VMEM((CHUNK, D), jnp.float32),
        pltpu.VMEM_SHARED((N_ACC, D), jnp.float32),
    ],
)
def _sc_degree(dst_hbm, zero_hbm, ones_hbm, out_hbm, dst_v, ones_v, acc_sh):
    cid = lax.axis_index("c")
    sid = lax.axis_index("s")
    wid = cid * NS + sid
    pltpu.sync_copy(dst_hbm.at[pl.ds(wid * NCHUNK, NCHUNK)], dst_v)
    pltpu.sync_copy(ones_hbm, ones_v)
    pltpu.sync_copy(zero_hbm.at[pl.ds(sid * RPT, RPT)],
                    acc_sh.at[pl.ds(sid * RPT, RPT)])
    plsc.subcore_barrier()

    def body(c, carry):
        pltpu.sync_copy(ones_v, acc_sh.at[dst_v.at[c]], add=True)
        return carry

    lax.fori_loop(0, NCHUNK, body, 0)

    plsc.subcore_barrier()
    pltpu.sync_copy(acc_sh.at[pl.ds(sid * RPT, RPT)],
                    out_hbm.at[cid].at[pl.ds(sid * RPT, RPT)])


def _elu(t):
    return jnp.where(t > 0, t, jnp.exp(t) - 1.0)


def _tc_pre(x, W1, dega, degb):
    def body(x_ref, w_ref, da_ref, db_ref, hp_ref, hp2_ref, dinv_ref):
        deg = da_ref[...] + db_ref[...] + 1.0
        dinv = lax.rsqrt(deg)
        z = jnp.dot(x_ref[...], w_ref[...], preferred_element_type=jnp.float32)
        hp = z * dinv
        hp_ref[...] = hp
        hp2_ref[...] = hp
        dinv_ref[...] = dinv

    return pl.pallas_call(
        body,
        grid=(N // RB,),
        in_specs=[
            pl.BlockSpec((RB, D), lambda i: (i, 0)),
            pl.BlockSpec((D, D), lambda i: (0, 0)),
            pl.BlockSpec((RB, 1), lambda i: (i, 0)),
            pl.BlockSpec((RB, 1), lambda i: (i, 0)),
        ],
        out_specs=[
            pl.BlockSpec((RB, D), lambda i: (i, 0)),
            pl.BlockSpec((RB, D), lambda i: (i, 0)),
            pl.BlockSpec((RB, 1), lambda i: (i, 0)),
        ],
        out_shape=[
            jax.ShapeDtypeStruct((N, D), jnp.float32),
            jax.ShapeDtypeStruct((N, D), jnp.float32),
            jax.ShapeDtypeStruct((N, 1), jnp.float32),
        ],
    )(x, W1, dega, degb)


def _tc_mid(agg0, agg1, hp, dinv, b, W):
    def body(a0_ref, a1_ref, hp_ref, dinv_ref, b_ref, w_ref,
             out_ref, out2_ref):
        dinv = dinv_ref[...]
        t = dinv * (a0_ref[...] + a1_ref[...] + hp_ref[...]) + b_ref[...]
        t = _elu(t)
        o = jnp.dot(t, w_ref[...], preferred_element_type=jnp.float32) * dinv
        out_ref[...] = o
        out2_ref[...] = o

    return pl.pallas_call(
        body,
        grid=(N // RB,),
        in_specs=[
            pl.BlockSpec((RB, D), lambda i: (i, 0)),
            pl.BlockSpec((RB, D), lambda i: (i, 0)),
            pl.BlockSpec((RB, D), lambda i: (i, 0)),
            pl.BlockSpec((RB, 1), lambda i: (i, 0)),
            pl.BlockSpec((1, D), lambda i: (0, 0)),
            pl.BlockSpec((D, D), lambda i: (0, 0)),
        ],
        out_specs=[
            pl.BlockSpec((RB, D), lambda i: (i, 0)),
            pl.BlockSpec((RB, D), lambda i: (i, 0)),
        ],
        out_shape=[
            jax.ShapeDtypeStruct((N, D), jnp.float32),
            jax.ShapeDtypeStruct((N, D), jnp.float32),
        ],
    )(agg0, agg1, hp, dinv, b, W)


def _tc_post(agg0, agg1, hp, dinv, b3, batch_col, Wl1, bl1, Wl2p, bl2p):
    nblk = N // RB

    def body(a0_ref, a1_ref, hp_ref, dinv_ref, b_ref, batch_ref,
             wl1_ref, bl1_ref, wl2_ref, bl2_ref, out_ref, pooled):
        i = pl.program_id(0)
        dinv = dinv_ref[...]
        t = dinv * (a0_ref[...] + a1_ref[...] + hp_ref[...]) + b_ref[...]
        t = _elu(t)
        seg = lax.broadcasted_iota(jnp.int32, (RB, G), 1)
        oh = (batch_ref[...] == seg).astype(jnp.float32)
        contrib = lax.dot_general(
            oh, t, (((0,), (0,)), ((), ())),
            preferred_element_type=jnp.float32)

        @pl.when(i == 0)
        def _():
            pooled[...] = contrib

        @pl.when(i > 0)
        def _():
            pooled[...] += contrib

        @pl.when(i == nblk - 1)
        def _():
            p = pooled[...]
            o1 = jnp.dot(p, wl1_ref[...],
                         preferred_element_type=jnp.float32) + bl1_ref[...]
            out_ref[...] = jnp.dot(
                o1, wl2_ref[...], preferred_element_type=jnp.float32) + bl2_ref[...]

    return pl.pallas_call(
        body,
        grid=(nblk,),
        in_specs=[
            pl.BlockSpec((RB, D), lambda i: (i, 0)),
            pl.BlockSpec((RB, D), lambda i: (i, 0)),
            pl.BlockSpec((RB, D), lambda i: (i, 0)),
            pl.BlockSpec((RB, 1), lambda i: (i, 0)),
            pl.BlockSpec((1, D), lambda i: (0, 0)),
            pl.BlockSpec((RB, 1), lambda i: (i, 0)),
            pl.BlockSpec((D, D), lambda i: (0, 0)),
            pl.BlockSpec((1, D), lambda i: (0, 0)),
            pl.BlockSpec((D, D), lambda i: (0, 0)),
            pl.BlockSpec((1, D), lambda i: (0, 0)),
        ],
        out_specs=pl.BlockSpec((G, D), lambda i: (0, 0)),
        out_shape=jax.ShapeDtypeStruct((G, D), jnp.float32),
        scratch_shapes=[pltpu.VMEM((G, D), jnp.float32)],
    )(agg0, agg1, hp, dinv, b3, batch_col, Wl1, bl1, Wl2p, bl2p)


def kernel(x, edge_index, batch, W1, b1, W2, b2, W3, b3, Wl1, bl1, Wl2, bl2):
    src = edge_index[0].astype(jnp.int32)
    dst = edge_index[1].astype(jnp.int32)
    pad = EPT_PAD * NW - E
    src_t = jnp.concatenate([src, jnp.zeros((pad,), jnp.int32)]).reshape(
        TOTCH, CHUNK)
    dst_t = jnp.concatenate([dst, jnp.full((pad,), N, jnp.int32)]).reshape(
        TOTCH, CHUNK)

    zeros_feat = jnp.zeros((N_ACC, D), jnp.float32)
    ones_rows = jnp.ones((CHUNK, D), jnp.float32)

    degp = _sc_degree(dst_t, zeros_feat, ones_rows)
    dega = degp[0, :N, 0:1]
    degb = degp[1, :N, 0:1]

    h1p, h1p2, dinv = _tc_pre(x, W1, dega, degb)

    agg = _sc_aggregate(h1p, h1p2, src_t, dst_t, zeros_feat)
    h2p, h2p2 = _tc_mid(agg[0, :N], agg[1, :N], h1p, dinv,
                        b1.reshape(1, D), W2)

    agg = _sc_aggregate(h2p, h2p2, src_t, dst_t, zeros_feat)
    h3p, h3p2 = _tc_mid(agg[0, :N], agg[1, :N], h2p, dinv,
                        b2.reshape(1, D), W3)

    agg = _sc_aggregate(h3p, h3p2, src_t, dst_t, zeros_feat)

    batch_col = batch.astype(jnp.int32).reshape(N, 1)
    Wl2p = jnp.zeros((D, D), jnp.float32).at[:, :2].set(Wl2)
    bl2p = jnp.zeros((1, D), jnp.float32).at[0, :2].set(bl2)
    out = _tc_post(agg[0, :N], agg[1, :N], h3p, dinv, b3.reshape(1, D),
                   batch_col, Wl1, bl1.reshape(1, D), Wl2p, bl2p)
    return out[:, :2]
